# baseline (speedup 1.0000x reference)
import functools

import jax
import jax.numpy as jnp
from jax import lax
from jax.experimental import pallas as pl
from jax.experimental.pallas import tpu as pltpu
from jax.experimental.pallas import tpu_sc as plsc

L = 16
NC = 2
NS = 16
NW = NC * NS
RCH = 128
UNROLL = 4


@functools.lru_cache(maxsize=None)
def _make_winners(M, B):
    BPW = B // NW
    NV = B // L
    NCHUNK = BPW // RCH
    CAP = BPW + L

    mesh = plsc.VectorSubcoreMesh(
        core_axis_name="c", subcore_axis_name="s", num_cores=NC, num_subcores=NS
    )

    @functools.partial(
        pl.kernel,
        out_type=(
            jax.ShapeDtypeStruct((NW, NCHUNK, RCH), jnp.int32),
            jax.ShapeDtypeStruct((NW, NCHUNK, RCH), jnp.int32),
        ),
        mesh=mesh,
        compiler_params=pltpu.CompilerParams(needs_layout_passes=False),
        scratch_types=[
            pltpu.VMEM((B,), jnp.int32),
            pltpu.VMEM((M,), jnp.int32),
            pltpu.VMEM((CAP,), jnp.int32),
            pltpu.VMEM((CAP,), jnp.int32),
            pltpu.VMEM((NCHUNK, RCH), jnp.int32),
            pltpu.VMEM((NCHUNK, RCH), jnp.int32),
        ],
    )
    def winners_kernel(idx_hbm, cidx_hbm, cpos_hbm, idxf, pos, cidx1, cpos1,
                       cidx2, cpos2):
        c = lax.axis_index("c")
        s = lax.axis_index("s")
        wid = s * NC + c
        lane = lax.iota(jnp.int32, L)

        pltpu.sync_copy(idx_hbm, idxf)

        def p1_body(k, _):
            for u in range(UNROLL):
                off = (k * UNROLL + u) * L
                v = idxf[pl.ds(off, L)]
                plsc.store_scatter(pos, [v], off + lane)
            return 0

        lax.fori_loop(0, NV // UNROLL, p1_body, 0)

        def fix_body(k, _):
            for u in range(UNROLL):
                off = (k * UNROLL + u) * L
                v = idxf[pl.ds(off, L)]
                b = off + lane
                p = plsc.load_gather(pos, [v])
                plsc.store_scatter(pos, [v], b, mask=p < b)
            return 0

        lax.fori_loop(0, NV // UNROLL, fix_body, 0)
        lax.fori_loop(0, NV // UNROLL, fix_body, 0)

        ebase = wid * BPW

        def ext_body(k, cursor):
            v = idxf[pl.ds(ebase + k * L, L)]
            b = ebase + k * L + lane
            p = plsc.load_gather(pos, [v])
            m = p == b
            plsc.store_compressed(cidx1.at[pl.ds(cursor, L)], v, mask=m)
            plsc.store_compressed(cpos1.at[pl.ds(cursor, L)], b, mask=m)
            return cursor + jnp.sum(m.astype(jnp.int32))

        n_win = lax.fori_loop(0, BPW // L, ext_body, jnp.int32(0))

        v0 = idxf[pl.ds(ebase, L)]
        p0 = plsc.load_gather(pos, [v0])
        is0 = lane == 0
        padi = jnp.sum(jnp.where(is0, v0, 0))
        padp = jnp.sum(jnp.where(is0, p0, 0))
        padiv = jnp.full((L,), padi, jnp.int32)
        padpv = jnp.full((L,), padp, jnp.int32)

        def pad_body(t, _):
            cidx1[pl.ds(n_win + t * L, L)] = padiv
            cpos1[pl.ds(n_win + t * L, L)] = padpv
            return 0

        lax.fori_loop(0, (BPW - n_win + L - 1) // L, pad_body, 0)

        for j in range(NCHUNK):
            for t in range(RCH // L):
                cidx2[j, pl.ds(t * L, L)] = cidx1[pl.ds(j * RCH + t * L, L)]
                cpos2[j, pl.ds(t * L, L)] = cpos1[pl.ds(j * RCH + t * L, L)]
        pltpu.sync_copy(cidx2, cidx_hbm.at[wid])
        pltpu.sync_copy(cpos2, cpos_hbm.at[wid])

    return winners_kernel


@functools.lru_cache(maxsize=None)
def _make_row_scatter(M, D, B):
    BPW = B // NW
    NCHUNK = BPW // RCH

    mesh = plsc.VectorSubcoreMesh(
        core_axis_name="c", subcore_axis_name="s", num_cores=NC, num_subcores=NS
    )

    @functools.partial(
        pl.kernel,
        out_type=(),
        mesh=mesh,
        compiler_params=pltpu.CompilerParams(needs_layout_passes=False),
        scratch_types=[
            pltpu.VMEM((NCHUNK, RCH), jnp.int32),
            pltpu.VMEM((NCHUNK, RCH), jnp.int32),
            pltpu.VMEM((RCH, D), jnp.float32),
            pltpu.VMEM((RCH, D), jnp.float32),
            pltpu.SemaphoreType.DMA,
            pltpu.SemaphoreType.DMA,
        ],
    )
    def row_scatter_kernel(cidx_hbm, cpos_hbm, val_hbm, out_hbm, cidxv, cposv,
                           rows0, rows1, gsem, ssem):
        c = lax.axis_index("c")
        s = lax.axis_index("s")
        wid = s * NC + c

        pltpu.sync_copy(cidx_hbm.at[wid], cidxv)
        pltpu.sync_copy(cpos_hbm.at[wid], cposv)

        rows = (rows0, rows1)
        gathers = [None] * NCHUNK
        scatters = [None] * NCHUNK
        gathers[0] = pltpu.async_copy(val_hbm.at[cposv.at[0]], rows[0], gsem)
        for j in range(1, NCHUNK):
            gathers[j - 1].wait()
            scatters[j - 1] = pltpu.async_copy(
                rows[(j - 1) % 2], out_hbm.at[cidxv.at[j - 1]], ssem
            )
            if j >= 2:
                scatters[j - 2].wait()
            gathers[j] = pltpu.async_copy(
                val_hbm.at[cposv.at[j]], rows[j % 2], gsem
            )
        gathers[NCHUNK - 1].wait()
        if NCHUNK >= 2:
            scatters[NCHUNK - 2].wait()
        scatters[NCHUNK - 1] = pltpu.async_copy(
            rows[(NCHUNK - 1) % 2], out_hbm.at[cidxv.at[NCHUNK - 1]], ssem
        )
        scatters[NCHUNK - 1].wait()

    return row_scatter_kernel


def kernel(mem, idx, val):
    M, D = mem.shape
    B = idx.shape[0]
    cidx, cpos = _make_winners(M, B)(idx)
    out_ref = jax.new_ref(mem)
    _make_row_scatter(M, D, B)(cidx, cpos, val, out_ref)
    return out_ref[...]


# speedup vs baseline: 1.0270x; 1.0270x over previous
import functools

import jax
import jax.numpy as jnp
from jax import lax
from jax.experimental import pallas as pl
from jax.experimental.pallas import tpu as pltpu
from jax.experimental.pallas import tpu_sc as plsc

L = 16
NC = 2
NS = 16
NW = NC * NS
RCH = 128
UNROLL = 4


@functools.lru_cache(maxsize=None)
def _make_winners(M, B):
    BPW = B // NW
    NV = B // L
    NCHUNK = BPW // RCH
    CAP = BPW + L

    mesh = plsc.VectorSubcoreMesh(
        core_axis_name="c", subcore_axis_name="s", num_cores=NC, num_subcores=NS
    )

    @functools.partial(
        pl.kernel,
        out_type=(
            jax.ShapeDtypeStruct((NW, NCHUNK, RCH), jnp.int32),
            jax.ShapeDtypeStruct((NW, NCHUNK, RCH), jnp.int32),
        ),
        mesh=mesh,
        compiler_params=pltpu.CompilerParams(needs_layout_passes=False),
        scratch_types=[
            pltpu.VMEM((B,), jnp.int32),
            pltpu.VMEM((M,), jnp.int32),
            pltpu.VMEM((CAP,), jnp.int32),
            pltpu.VMEM((CAP,), jnp.int32),
            pltpu.VMEM((NCHUNK, RCH), jnp.int32),
            pltpu.VMEM((NCHUNK, RCH), jnp.int32),
        ],
    )
    def winners_kernel(idx_hbm, cidx_hbm, cpos_hbm, idxf, pos, cidx1, cpos1,
                       cidx2, cpos2):
        c = lax.axis_index("c")
        s = lax.axis_index("s")
        wid = s * NC + c
        lane = lax.iota(jnp.int32, L)

        pltpu.sync_copy(idx_hbm, idxf)

        def p1_body(k, _):
            for u in range(UNROLL):
                off = (k * UNROLL + u) * L
                v = idxf[pl.ds(off, L)]
                plsc.store_scatter(pos, [v], off + lane)
            return 0

        lax.fori_loop(0, NV // UNROLL, p1_body, 0)

        def fix_body(k, _):
            for u in range(UNROLL):
                off = (k * UNROLL + u) * L
                v = idxf[pl.ds(off, L)]
                b = off + lane
                p = plsc.load_gather(pos, [v])
                plsc.store_scatter(pos, [v], b, mask=p < b)
            return 0

        lax.fori_loop(0, NV // UNROLL, fix_body, 0)
        lax.fori_loop(0, NV // UNROLL, fix_body, 0)

        ebase = wid * BPW

        def ext_body(k, cursor):
            v = idxf[pl.ds(ebase + k * L, L)]
            b = ebase + k * L + lane
            p = plsc.load_gather(pos, [v])
            m = p == b
            plsc.store_compressed(cidx1.at[pl.ds(cursor, L)], v, mask=m)
            plsc.store_compressed(cpos1.at[pl.ds(cursor, L)], b, mask=m)
            return cursor + jnp.sum(m.astype(jnp.int32))

        n_win = lax.fori_loop(0, BPW // L, ext_body, jnp.int32(0))

        v0 = idxf[pl.ds(ebase, L)]
        p0 = plsc.load_gather(pos, [v0])
        is0 = lane == 0
        padi = jnp.sum(jnp.where(is0, v0, 0))
        padp = jnp.sum(jnp.where(is0, p0, 0))
        padiv = jnp.full((L,), padi, jnp.int32)
        padpv = jnp.full((L,), padp, jnp.int32)

        def pad_body(t, _):
            cidx1[pl.ds(n_win + t * L, L)] = padiv
            cpos1[pl.ds(n_win + t * L, L)] = padpv
            return 0

        lax.fori_loop(0, (BPW - n_win + L - 1) // L, pad_body, 0)

        for j in range(NCHUNK):
            for t in range(RCH // L):
                cidx2[j, pl.ds(t * L, L)] = cidx1[pl.ds(j * RCH + t * L, L)]
                cpos2[j, pl.ds(t * L, L)] = cpos1[pl.ds(j * RCH + t * L, L)]
        pltpu.sync_copy(cidx2, cidx_hbm.at[wid])
        pltpu.sync_copy(cpos2, cpos_hbm.at[wid])

    return winners_kernel


@functools.lru_cache(maxsize=None)
def _make_row_scatter(M, D, B):
    BPW = B // NW
    NCHUNK = BPW // RCH

    mesh = plsc.VectorSubcoreMesh(
        core_axis_name="c", subcore_axis_name="s", num_cores=NC, num_subcores=NS
    )

    @functools.partial(
        pl.kernel,
        out_type=(),
        mesh=mesh,
        compiler_params=pltpu.CompilerParams(needs_layout_passes=False),
        scratch_types=[
            pltpu.VMEM((NCHUNK, RCH), jnp.int32),
            pltpu.VMEM((NCHUNK, RCH), jnp.int32),
        ]
        + [pltpu.VMEM((RCH, D), jnp.float32) for _ in range(NCHUNK)]
        + [
            pltpu.SemaphoreType.DMA,
            pltpu.SemaphoreType.DMA,
        ],
    )
    def row_scatter_kernel(cidx_hbm, cpos_hbm, val_hbm, out_hbm, cidxv, cposv,
                           *rest):
        rows = rest[:NCHUNK]
        gsem, ssem = rest[NCHUNK:]
        c = lax.axis_index("c")
        s = lax.axis_index("s")
        wid = s * NC + c

        pltpu.sync_copy(cidx_hbm.at[wid], cidxv)
        pltpu.sync_copy(cpos_hbm.at[wid], cposv)

        gathers = [
            pltpu.async_copy(val_hbm.at[cposv.at[j]], rows[j], gsem)
            for j in range(NCHUNK)
        ]
        scatters = []
        for j in range(NCHUNK):
            gathers[j].wait()
            scatters.append(
                pltpu.async_copy(rows[j], out_hbm.at[cidxv.at[j]], ssem)
            )
        for sc_ in scatters:
            sc_.wait()

    return row_scatter_kernel


def kernel(mem, idx, val):
    M, D = mem.shape
    B = idx.shape[0]
    out_ref = jax.new_ref(mem)
    cidx, cpos = _make_winners(M, B)(idx)
    _make_row_scatter(M, D, B)(cidx, cpos, val, out_ref)
    return out_ref[...]
